# baseline (speedup 1.0000x reference)
import functools

import jax
import jax.numpy as jnp
from jax import lax
from jax.experimental import pallas as pl
from jax.experimental.pallas import tpu as pltpu

N = 10000
E = 160000
NP = 10240
EP = 163840


def _row_call(fn, out_shapes, row_args, const_args=(), block=2048):
    R = row_args[0].shape[0]
    while R % block:
        block //= 2
    grid = (R // block,)

    def row_spec(a):
        nd = a.ndim
        return pl.BlockSpec((block,) + a.shape[1:],
                            lambda i, nd=nd: (i,) + (0,) * (nd - 1))

    def const_spec(a):
        nd = a.ndim
        return pl.BlockSpec(a.shape, lambda i, nd=nd: (0,) * nd)

    in_specs = [row_spec(a) for a in row_args] + [const_spec(c) for c in const_args]
    out_specs = tuple(pl.BlockSpec((block,) + s[1:],
                                   lambda i, nd=len(s): (i,) + (0,) * (nd - 1))
                      for s in out_shapes)
    out_shape = tuple(jax.ShapeDtypeStruct(s, jnp.float32) for s in out_shapes)
    n_in = len(row_args) + len(const_args)

    def body(*refs):
        ins = [r[...] for r in refs[:n_in]]
        outs = fn(*ins)
        if not isinstance(outs, (tuple, list)):
            outs = (outs,)
        for oref, val in zip(refs[n_in:], outs):
            oref[...] = val

    res = pl.pallas_call(
        body, grid=grid, in_specs=in_specs, out_specs=out_specs,
        out_shape=out_shape)(*row_args, *const_args)
    return res


def _row_reduce_sum(fn, row_args, const_args=(), block=2048):
    R = row_args[0].shape[0]
    grid = (R // block,)

    def row_spec(a):
        nd = a.ndim
        return pl.BlockSpec((block,) + a.shape[1:],
                            lambda i, nd=nd: (i,) + (0,) * (nd - 1))

    def const_spec(a):
        nd = a.ndim
        return pl.BlockSpec(a.shape, lambda i, nd=nd: (0,) * nd)

    in_specs = [row_spec(a) for a in row_args] + [const_spec(c) for c in const_args]
    n_in = len(row_args) + len(const_args)

    def body(*refs):
        i = pl.program_id(0)
        out = refs[-1]
        @pl.when(i == 0)
        def _():
            out[...] = jnp.zeros_like(out)
        ins = [r[...] for r in refs[:n_in]]
        out[...] += jnp.sum(fn(*ins)).reshape(1, 1)

    return pl.pallas_call(
        body, grid=grid, in_specs=in_specs,
        out_specs=pl.BlockSpec((1, 1), lambda i: (0, 0)),
        out_shape=jax.ShapeDtypeStruct((1, 1), jnp.float32))(*row_args, *const_args)


def _qmul(a, b):
    aw, ax, ay, az = a[:, 0], a[:, 1], a[:, 2], a[:, 3]
    bw, bx, by, bz = b[:, 0], b[:, 1], b[:, 2], b[:, 3]
    return jnp.stack([
        aw * bw - ax * bx - ay * by - az * bz,
        aw * bx + ax * bw + ay * bz - az * by,
        aw * by - ax * bz + ay * bw + az * bx,
        aw * bz + ax * by - ay * bx + az * bw], axis=1)


def _qmul_inv_a(a, b):
    ai = jnp.stack([a[:, 0], -a[:, 1], -a[:, 2], -a[:, 3]], axis=1)
    return _qmul(ai, b)


def _l2n(v):
    return v / (jnp.sqrt(jnp.sum(v * v, axis=1, keepdims=True)) + 1e-8)


def _node_mm(x, W1, W2):
    F = W1.shape[1]
    return _row_call(
        lambda xb, w1, w2: (jnp.dot(xb, w1, preferred_element_type=jnp.float32),
                            jnp.dot(xb, w2, preferred_element_type=jnp.float32)),
        [(x.shape[0], F), (x.shape[0], F)], [x], [W1, W2])


def _node_finish(agg, cnt):
    return _row_call(
        lambda a, c: jax.nn.relu(a / jnp.maximum(c, 1.0)),
        [agg.shape], [agg, cnt])[0]


def _edge_p(ea_list, W3, b):
    F = W3.shape[1]
    return _row_call(
        lambda *args: jnp.dot(jnp.concatenate(args[:-2], axis=1), args[-2],
                              preferred_element_type=jnp.float32) + args[-1],
        [(ea_list[0].shape[0], F)], list(ea_list), [W3, b.reshape(1, -1)])[0]


def _relu_rows(a):
    return _row_call(lambda v: jax.nn.relu(v), [a.shape], [a])[0]


def _conv_scatter(xW1, xW2, p, rowi, coli, m, n_out, mask_mult):
    e = xW1[rowi] + xW2[coli] + p
    if mask_mult:
        e = e * m[:, None]
    agg = jnp.zeros((n_out, e.shape[1]), jnp.float32).at[coli].add(e)
    cnt = jnp.zeros((n_out,), jnp.float32).at[coli].add(m)
    return e, agg, cnt.reshape(-1, 1)


def _gather_rows(tab, idx):
    return tab[idx]


def _scatter_rows_add(base, idx, delta):
    return base.at[idx].add(delta)


def _gat_scores(Wg, a_s, a_d, x, rowi, coli, m, n_out):
    h = (x @ Wg)[:, 0]
    e = jax.nn.leaky_relu(a_s * h[rowi] + a_d * h[coli], 0.2)
    e = jnp.where(m > 0, e, -1e9)
    mx = jnp.full((n_out,), -1e9, jnp.float32).at[coli].max(e)
    ex = jnp.exp(e - mx[coli]) * m
    den = jnp.zeros((n_out,), jnp.float32).at[coli].add(ex)
    alpha = ex / jnp.maximum(den[coli], 1e-9)
    gat = jnp.zeros((n_out,), jnp.float32).at[coli].add(alpha * h[rowi])
    return jnp.tanh(gat)


def _pool_remap(perm, k, rowi, coli, m, n_nodes):
    kept = jnp.zeros((n_nodes,), bool).at[perm].set(True)
    nid = jnp.zeros((n_nodes,), jnp.int32).at[perm].set(jnp.arange(k, dtype=jnp.int32))
    valid = kept[jnp.minimum(rowi, n_nodes - 1)] & kept[jnp.minimum(coli, n_nodes - 1)] & (m > 0)
    row2 = jnp.where(valid, nid[jnp.minimum(rowi, n_nodes - 1)], 0)
    col2 = jnp.where(valid, nid[jnp.minimum(coli, n_nodes - 1)], 0)
    fm = valid.astype(jnp.float32)
    return row2, col2, fm


def kernel(x_org, edge_index, edge_attr, gt_q, beta, params):
    p_ = params
    rowi = jnp.concatenate([edge_index[0], jnp.full((EP - E,), N, jnp.int32)])
    coli = jnp.concatenate([edge_index[1], jnp.full((EP - E,), N, jnp.int32)])
    ones_m = jnp.concatenate([jnp.ones((E,), jnp.float32), jnp.zeros((EP - E,), jnp.float32)])
    ea8 = jnp.pad(edge_attr, ((0, EP - E), (0, 0)))
    x_org_p = jnp.pad(x_org, ((0, NP - N), (0, 0)))
    gt_q_p = jnp.pad(gt_q, ((0, NP - N), (0, 0)))
    k1, k2 = N // 2, N // 4
    K1P, K2P = 5120, 2560

    x0r = _gather_rows(x_org_p, rowi)
    x0c = _gather_rows(x_org_p, coli)
    eam = _row_call(
        lambda a, bq, e4: _l2n(_qmul(e4, _qmul_inv_a(a, bq))),
        [(EP, 4)], [x0r, x0c, ea8[:, :4]])[0]

    W, b = p_['c1W'], p_['c1b']
    xW1, xW2 = _node_mm(x_org_p, W[0:4], W[4:8])
    p1 = _edge_p([eam], W[8:12], b)
    e1p, agg, cnt = _conv_scatter(xW1, xW2, p1, rowi, coli, ones_m, NP, False)
    deg = cnt
    x1 = _node_finish(agg, deg)
    e1r = _relu_rows(e1p)

    W, b = p_['c2W'], p_['c2b']
    xW1, xW2 = _node_mm(x1, W[0:32], W[32:64])
    p2 = _edge_p([eam, e1r], W[64:100], b)
    e2p, agg, _ = _conv_scatter(xW1, xW2, p2, rowi, coli, ones_m, NP, False)
    x2 = _node_finish(agg, deg)
    e2r = _relu_rows(e2p)

    W, b = p_['c3W'], p_['c3b']
    xc = jnp.concatenate([x2, x1], axis=1)
    xW1, xW2 = _node_mm(xc, W[0:64], W[64:128])
    p3 = _edge_p([e2r, e1r], W[128:192], b)
    e3p, agg, _ = _conv_scatter(xW1, xW2, p3, rowi, coli, ones_m, NP, False)
    x3 = _node_finish(agg, deg)
    e3r = _relu_rows(e3p)

    W, b = p_['spW'], p_['spb']
    xW1, xW2 = _node_mm(x3, W[0:32], W[32:64])
    psp = _edge_p([e3r], W[64:96], b)
    esp_p, agg, _ = _conv_scatter(xW1, xW2, psp, rowi, coli, ones_m, NP, False)
    xs1 = _node_finish(agg, deg)
    es1r = _relu_rows(esp_p)

    score1 = _gat_scores(p_['g1Wg'], p_['g1as'], p_['g1ad'], xs1, rowi, coli, ones_m, NP)
    vals1, perm1 = lax.top_k(score1[:N], k1)
    row2, col2, fm1 = _pool_remap(perm1, k1, rowi, coli, ones_m, N)
    xs1g = _gather_rows(xs1, perm1)
    xs1p = jnp.pad(xs1g * vals1[:, None], ((0, K1P - k1), (0, 0)))

    W, b = p_['s1W'], p_['s1b']
    xW1, xW2 = _node_mm(xs1p, W[0:64], W[64:128])
    es1m = _row_call(lambda ev, mv: ev * mv, [(EP, 64)], [es1r, fm1.reshape(-1, 1)])[0]
    ps1 = _edge_p([es1m], W[128:192], b)
    es2_p, agg, cnt1 = _conv_scatter(xW1, xW2, ps1, row2, col2, fm1, K1P, True)
    xs2 = _node_finish(agg, cnt1)
    es2r = _relu_rows(es2_p)

    score2 = _gat_scores(p_['g2Wg'], p_['g2as'], p_['g2ad'], xs2, row2, col2, fm1, K1P)
    vals2, perm2 = lax.top_k(score2[:k1], k2)
    row3, col3, fm2 = _pool_remap(perm2, k2, row2, col2, fm1, k1)
    xs2g = _gather_rows(xs2, perm2)
    xssp = jnp.pad(xs2g * vals2[:, None], ((0, K2P - k2), (0, 0)))

    W, b = p_['ssW'], p_['ssb']
    xW1, xW2 = _node_mm(xssp, W[0:64], W[64:128])
    es2m = _row_call(lambda ev, mv: ev * mv, [(EP, 64)], [es2r, fm2.reshape(-1, 1)])[0]
    pss1 = _edge_p([es2m], W[128:192], b)
    ess1_p, agg, cnt2 = _conv_scatter(xW1, xW2, pss1, row3, col3, fm2, K2P, True)
    xss1 = _node_finish(agg, cnt2)
    ess1r = _relu_rows(ess1_p)

    xW1, xW2 = _node_mm(xss1, W[0:64], W[64:128])
    pss2 = _edge_p([ess1r], W[128:192], b)
    ess2_p, agg, _ = _conv_scatter(xW1, xW2, pss2, row3, col3, fm2, K2P, True)
    xss2 = _node_finish(agg, cnt2)

    xs2m = _scatter_rows_add(xs2, perm2, xss2[:k2])

    W, b = p_['s2W'], p_['s2b']
    xW1, xW2 = _node_mm(xs2m, W[0:64], W[64:128])
    ps2 = _edge_p([es2r], W[128:192], b)
    es3_p, agg, _ = _conv_scatter(xW1, xW2, ps2, row2, col2, fm1, K1P, True)
    xs3 = _node_finish(agg, cnt1)

    x3b = _scatter_rows_add(x3, perm1, xs3[:k1])

    W, b = p_['c4W'], p_['c4b']
    xc4 = jnp.concatenate([x3b, x2], axis=1)
    xW1, xW2 = _node_mm(xc4, W[0:64], W[64:128])
    p4 = _edge_p([e3r, e2r], W[128:192], b)
    e4p, agg, _ = _conv_scatter(xW1, xW2, p4, rowi, coli, ones_m, NP, False)
    x4 = _node_finish(agg, deg)
    e4r = _relu_rows(e4p)

    xc5 = jnp.concatenate([x4, x3b], axis=1)
    xW1, xW2 = _node_mm(xc5, W[0:64], W[64:128])
    p5 = _edge_p([e4r, e3r], W[128:192], b)
    e5p, agg, _ = _conv_scatter(xW1, xW2, p5, rowi, coli, ones_m, NP, False)
    x5 = _node_finish(agg, deg)
    e5r = _relu_rows(e5p)

    xq = _row_call(
        lambda xv, xo, w, bb: _l2n(_qmul(jnp.dot(xv, w, preferred_element_type=jnp.float32) + bb, xo)),
        [(NP, 4)], [x5, x_org_p], [p_['l1W'], p_['l1b'].reshape(1, -1)])[0]

    gqr = _gather_rows(gt_q_p, rowi)
    gqc = _gather_rows(gt_q_p, coli)
    xr = _gather_rows(xq, rowi)
    xc_ = _gather_rows(xq, coli)

    def loss_fn(gr, gc, ar, ac, mv, bt):
        qa = _qmul_inv_a(gr, gc)
        qb = _qmul_inv_a(ar, ac)
        lv = _l2n(_qmul_inv_a(qa, qb))
        d = jnp.abs(jnp.stack([lv[:, 0] - 1.0, lv[:, 1], lv[:, 2], lv[:, 3]], axis=1))
        bb = jnp.maximum(bt[0, 0], 1e-6)
        contrib = jnp.where(d < bb, 0.5 * d * d / bb, d - 0.5 * bb)
        return contrib * mv

    loss_sum = _row_reduce_sum(loss_fn, [gqr, gqc, xr, xc_, ones_m.reshape(-1, 1)],
                               [beta.reshape(1, 1)])
    loss1 = loss_sum[0, 0] / (E * 4.0)

    return (xq[:N], loss1, beta,
            (x1[:N], x2[:N], x3b[:N], x4[:N], x5[:N]),
            (e1r[:E], e2r[:E], e3r[:E], e4r[:E], e5r[:E]))
